# Initial kernel scaffold; baseline (speedup 1.0000x reference)
import jax
import jax.numpy as jnp
from jax.experimental import pallas as pl


def kernel(input_ids, video_features, token_type_ids, word_table, tok_table, wfc_ln1_w, wfc_ln1_b, wfc_W, wfc_b, wfc_ln2_w, wfc_ln2_b, vid_ln1_w, vid_ln1_b, vid_W, vid_b, vid_ln2_w, vid_ln2_b, final_ln_w, final_ln_b, pe):
    raise NotImplementedError("write your pallas kernel here")


# speedup vs baseline: 2.6041x; 2.6041x over previous
import functools

import jax
import jax.numpy as jnp
from jax import lax
from jax.experimental import pallas as pl
from jax.experimental.pallas import tpu as pltpu
from jax.experimental.pallas import tpu_sc as plsc

_EPS = 1e-12
_TB = 256
_WPAD = 384
_WMAIN = 256


def _ln(x, w, b):
    u = jnp.mean(x, axis=-1, keepdims=True)
    s = jnp.mean((x - u) ** 2, axis=-1, keepdims=True)
    return w * ((x - u) * lax.rsqrt(s + _EPS)) + b


def _sc_gather(word_table, tail128, ids):
    n = ids.size
    nw = 32
    bpw = n // nw
    ch = 128
    nch = bpw // ch

    mesh = plsc.VectorSubcoreMesh(core_axis_name="c", subcore_axis_name="s")

    @functools.partial(
        pl.kernel,
        mesh=mesh,
        out_type=jax.ShapeDtypeStruct((n, _WPAD), jnp.float32),
        scratch_types=[
            pltpu.VMEM((nch, ch), jnp.int32),
            pltpu.VMEM((ch, _WPAD), jnp.float32),
            pltpu.VMEM((ch, _WPAD), jnp.float32),
            pltpu.SemaphoreType.DMA,
            pltpu.SemaphoreType.DMA,
        ],
    )
    def k(table_hbm, tail_hbm, idx_hbm, out_hbm, idx_v, rows0, rows1,
          sem0, sem1):
        wid = lax.axis_index("s") * 2 + lax.axis_index("c")
        base = wid * bpw
        pltpu.sync_copy(idx_hbm.at[pl.ds(wid * nch, nch)], idx_v)
        bufs = (rows0, rows1)
        sems = (sem0, sem1)

        def fire(j):
            return (
                pltpu.async_copy(
                    table_hbm.at[idx_v.at[j], pl.ds(0, _WMAIN)],
                    bufs[j % 2].at[:, pl.ds(0, _WMAIN)], sems[j % 2]),
                pltpu.async_copy(
                    tail_hbm.at[idx_v.at[j]],
                    bufs[j % 2].at[:, pl.ds(_WMAIN, _WPAD - _WMAIN)],
                    sems[j % 2]),
            )

        pending = fire(0)
        for j in range(nch):
            cur = pending
            if j + 1 < nch:
                pending = fire(j + 1)
            for cp in cur:
                cp.wait()
            pltpu.sync_copy(bufs[j % 2], out_hbm.at[pl.ds(base + j * ch, ch)])

    return k(word_table, tail128, ids.reshape(n // ch, ch))


def _tc_body(we_ref, vf_ref, tt_ref, wfcW_ref, vidW_ref, tok_ref, pe_ref,
             wl1w, wl1b, wfcb, wl2w, wl2b,
             vl1w, vl1b, vidb, vl2w, vl2b,
             flw, flb, out_ref):
    x = we_ref[...]
    wvec = 300.0
    mask = lax.broadcasted_iota(jnp.int32, x.shape, 1) < 300
    u = jnp.sum(jnp.where(mask, x, 0.0), axis=-1, keepdims=True) / wvec
    dx = jnp.where(mask, x - u, 0.0)
    s = jnp.sum(dx * dx, axis=-1, keepdims=True) / wvec
    we = wl1w[...] * (dx * lax.rsqrt(s + _EPS)) + wl1b[...]
    hw = jnp.dot(we.astype(jnp.bfloat16), wfcW_ref[...],
                 preferred_element_type=jnp.float32) + wfcb[...]
    hw = _ln(jnp.maximum(hw, 0.0), wl2w[...], wl2b[...])
    vf = _ln(vf_ref[...], vl1w[...], vl1b[...])
    hv = jnp.dot(vf.astype(jnp.bfloat16), vidW_ref[...],
                 preferred_element_type=jnp.float32) + vidb[...]
    hv = _ln(jnp.maximum(hv, 0.0), vl2w[...], vl2b[...])
    ids = tt_ref[...]
    tt = jnp.where(ids == 0, tok_ref[0:1, :],
                   jnp.where(ids == 1, tok_ref[1:2, :], tok_ref[2:3, :]))
    emb = hw + hv + tt + pe_ref[...]
    out_ref[...] = _ln(emb, flw[...], flb[...])


def _tc_fused(we, vf, ttc, wfcW, vidW, tok_table, pe_t, vparams):
    n = we.shape[0]
    vid_feat = vf.shape[1]
    hid = vidW.shape[1]
    grid = n // _TB

    def tok_block(i):
        return (i, 0)

    def whole(i):
        return (0, 0)

    in_specs = [
        pl.BlockSpec((_TB, _WPAD), tok_block),
        pl.BlockSpec((_TB, vid_feat), tok_block),
        pl.BlockSpec((_TB, 1), tok_block),
        pl.BlockSpec(wfcW.shape, whole),
        pl.BlockSpec(vidW.shape, whole),
        pl.BlockSpec(tok_table.shape, whole),
        pl.BlockSpec(pe_t.shape, whole),
    ] + [pl.BlockSpec(p.shape, whole) for p in vparams]

    return pl.pallas_call(
        _tc_body,
        grid=(grid,),
        in_specs=in_specs,
        out_specs=pl.BlockSpec((_TB, hid), tok_block),
        out_shape=jax.ShapeDtypeStruct((n, hid), jnp.float32),
    )(we, vf, ttc, wfcW, vidW, tok_table, pe_t, *vparams)


def kernel(input_ids, video_features, token_type_ids, word_table, tok_table,
           wfc_ln1_w, wfc_ln1_b, wfc_W, wfc_b, wfc_ln2_w, wfc_ln2_b,
           vid_ln1_w, vid_ln1_b, vid_W, vid_b, vid_ln2_w, vid_ln2_b,
           final_ln_w, final_ln_b, pe):
    b, l = input_ids.shape
    n = b * l
    hid = vid_W.shape[1]
    wvec = word_table.shape[1]

    ids = input_ids.reshape(n).astype(jnp.int32)
    tail128 = jnp.pad(word_table[:, _WMAIN:], ((0, 0), (0, _WPAD - wvec)))
    we_raw = _sc_gather(word_table, tail128, ids)

    vf = video_features.reshape(n, video_features.shape[2])
    ttc = token_type_ids.reshape(n, 1).astype(jnp.int32)
    pe_t = jnp.tile(pe[:l], (_TB // l, 1))

    row = lambda p: p.reshape(1, -1)
    pad_w = lambda p: jnp.pad(p.reshape(1, -1), ((0, 0), (0, _WPAD - wvec)))
    wfcW_pad = jnp.pad(wfc_W, ((0, _WPAD - wvec), (0, 0)))
    vparams = (pad_w(wfc_ln1_w), pad_w(wfc_ln1_b), row(wfc_b),
               row(wfc_ln2_w), row(wfc_ln2_b),
               row(vid_ln1_w), row(vid_ln1_b), row(vid_b),
               row(vid_ln2_w), row(vid_ln2_b),
               row(final_ln_w), row(final_ln_b))

    out = _tc_fused(we_raw, vf, ttc,
                    wfcW_pad.astype(jnp.bfloat16), vid_W.astype(jnp.bfloat16),
                    tok_table, pe_t, vparams)
    return out.reshape(b, l, hid)


# speedup vs baseline: 2.7255x; 1.0467x over previous
import functools

import jax
import jax.numpy as jnp
from jax import lax
from jax.experimental import pallas as pl
from jax.experimental.pallas import tpu as pltpu
from jax.experimental.pallas import tpu_sc as plsc

_EPS = 1e-12
_TB = 256
_WPAD = 384
_WMAIN = 256


def _sc_gather(word_table, tail128, ids):
    n = ids.size
    nw = 32
    bpw = n // nw
    ch = 128
    nch = bpw // ch

    mesh = plsc.VectorSubcoreMesh(core_axis_name="c", subcore_axis_name="s")

    @functools.partial(
        pl.kernel,
        mesh=mesh,
        out_type=jax.ShapeDtypeStruct((n, _WPAD), jnp.float32),
        scratch_types=[
            pltpu.VMEM((nch, ch), jnp.int32),
            pltpu.VMEM((ch, _WPAD), jnp.float32),
            pltpu.VMEM((ch, _WPAD), jnp.float32),
            pltpu.SemaphoreType.DMA,
            pltpu.SemaphoreType.DMA,
        ],
    )
    def k(table_hbm, tail_hbm, idx_hbm, out_hbm, idx_v, rows0, rows1,
          sem0, sem1):
        wid = lax.axis_index("s") * 2 + lax.axis_index("c")
        base = wid * bpw
        pltpu.sync_copy(idx_hbm.at[pl.ds(wid * nch, nch)], idx_v)
        bufs = (rows0, rows1)
        sems = (sem0, sem1)

        def fire(j):
            return (
                pltpu.async_copy(
                    table_hbm.at[idx_v.at[j], pl.ds(0, _WMAIN)],
                    bufs[j % 2].at[:, pl.ds(0, _WMAIN)], sems[j % 2]),
                pltpu.async_copy(
                    tail_hbm.at[idx_v.at[j]],
                    bufs[j % 2].at[:, pl.ds(_WMAIN, _WPAD - _WMAIN)],
                    sems[j % 2]),
            )

        pending = fire(0)
        for j in range(nch):
            cur = pending
            if j + 1 < nch:
                pending = fire(j + 1)
            for cp in cur:
                cp.wait()
            pltpu.sync_copy(bufs[j % 2], out_hbm.at[pl.ds(base + j * ch, ch)])

    return k(word_table, tail128, ids.reshape(n // ch, ch))


def _ln(x, w, b, dim):
    u = jnp.sum(x, axis=-1, keepdims=True) * (1.0 / dim)
    s = jnp.sum(x * x, axis=-1, keepdims=True) * (1.0 / dim) - u * u
    return w * ((x - u) * lax.rsqrt(s + _EPS)) + b


def _tc_body(we_ref, vf_ref, tt_ref, wfcW_ref, vidW_ref, tok_ref, pe_ref,
             wcs, wc, wl2w, wl2b,
             vcs, vc, vl2w, vl2b,
             flw, flb, out_ref):
    x = we_ref[...]
    inv = 1.0 / 300.0
    u = jnp.sum(x, axis=-1, keepdims=True) * inv
    mask = lax.broadcasted_iota(jnp.int32, x.shape, 1) < 300
    s = jnp.sum(jnp.where(mask, x - u, 0.0) ** 2, axis=-1, keepdims=True) * inv
    mm = jnp.dot(x.astype(jnp.bfloat16), wfcW_ref[...],
                 preferred_element_type=jnp.float32)
    hw = (mm - u * wcs[...]) * lax.rsqrt(s + _EPS) + wc[...]
    hw = _ln(jnp.maximum(hw, 0.0), wl2w[...], wl2b[...], 768.0)
    v = vf_ref[...]
    inv_v = 1.0 / 1024.0
    uv = jnp.sum(v, axis=-1, keepdims=True) * inv_v
    sv = jnp.sum(v * v, axis=-1, keepdims=True) * inv_v - uv * uv
    mv = jnp.dot(v.astype(jnp.bfloat16), vidW_ref[...],
                 preferred_element_type=jnp.float32)
    hv = (mv - uv * vcs[...]) * lax.rsqrt(sv + _EPS) + vc[...]
    hv = _ln(jnp.maximum(hv, 0.0), vl2w[...], vl2b[...], 768.0)
    ids = tt_ref[0]
    ids2 = ids.reshape(_TB, 1)
    tt = jnp.where(ids2 == 0, tok_ref[0:1, :],
                   jnp.where(ids2 == 1, tok_ref[1:2, :], tok_ref[2:3, :]))
    emb = hw + hv + tt + pe_ref[...]
    out_ref[...] = _ln(emb, flw[...], flb[...], 768.0)


def _tc_fused(we, vf, ttc, wfcW, vidW, tok_table, pe_t, vparams):
    n = we.shape[0]
    vid_feat = vf.shape[1]
    hid = vidW.shape[1]
    grid = n // _TB

    def tok_block(i):
        return (i, 0)

    def tok_block3(i):
        return (i, 0, 0)

    def whole(i):
        return (0, 0)

    in_specs = [
        pl.BlockSpec((_TB, _WPAD), tok_block),
        pl.BlockSpec((_TB, vid_feat), tok_block),
        pl.BlockSpec((1, 1, _TB), tok_block3),
        pl.BlockSpec(wfcW.shape, whole),
        pl.BlockSpec(vidW.shape, whole),
        pl.BlockSpec(tok_table.shape, whole),
        pl.BlockSpec(pe_t.shape, whole),
    ] + [pl.BlockSpec(p.shape, whole) for p in vparams]

    return pl.pallas_call(
        _tc_body,
        grid=(grid,),
        in_specs=in_specs,
        out_specs=pl.BlockSpec((_TB, hid), tok_block),
        out_shape=jax.ShapeDtypeStruct((n, hid), jnp.float32),
    )(we, vf, ttc, wfcW, vidW, tok_table, pe_t, *vparams)


def kernel(input_ids, video_features, token_type_ids, word_table, tok_table,
           wfc_ln1_w, wfc_ln1_b, wfc_W, wfc_b, wfc_ln2_w, wfc_ln2_b,
           vid_ln1_w, vid_ln1_b, vid_W, vid_b, vid_ln2_w, vid_ln2_b,
           final_ln_w, final_ln_b, pe):
    b, l = input_ids.shape
    n = b * l
    hid = vid_W.shape[1]
    wvec = word_table.shape[1]

    ids = input_ids.reshape(n).astype(jnp.int32)
    tail128 = jnp.pad(word_table[:, _WMAIN:], ((0, 0), (0, _WPAD - wvec)))
    we_raw = _sc_gather(word_table, tail128, ids)

    vf = video_features.reshape(n, video_features.shape[2])
    ttc = token_type_ids.astype(jnp.int32).reshape(n // _TB, 1, _TB)
    pe_t = jnp.tile(pe[:l], (_TB // l, 1))

    row = lambda p: p.reshape(1, -1)
    wfcWs = wfc_ln1_w[:, None] * wfc_W
    wfcWs_pad = jnp.pad(wfcWs, ((0, _WPAD - wvec), (0, 0)))
    w_colsum = row(jnp.sum(wfcWs, axis=0))
    w_const = row(wfc_ln1_b @ wfc_W + wfc_b)
    vidWs = vid_ln1_w[:, None] * vid_W
    v_colsum = row(jnp.sum(vidWs, axis=0))
    v_const = row(vid_ln1_b @ vid_W + vid_b)

    vparams = (w_colsum, w_const, row(wfc_ln2_w), row(wfc_ln2_b),
               v_colsum, v_const, row(vid_ln2_w), row(vid_ln2_b),
               row(final_ln_w), row(final_ln_b))

    out = _tc_fused(we_raw, vf, ttc,
                    wfcWs_pad.astype(jnp.bfloat16), vidWs.astype(jnp.bfloat16),
                    tok_table, pe_t, vparams)
    return out.reshape(b, l, hid)


# speedup vs baseline: 3.2515x; 1.1930x over previous
import functools

import jax
import jax.numpy as jnp
from jax import lax
from jax.experimental import pallas as pl
from jax.experimental.pallas import tpu as pltpu
from jax.experimental.pallas import tpu_sc as plsc

_EPS = 1e-12
_TB = 256
_NW = 32
_CHT = 8192


def _sc_gather_t(wt_t, ids):
    nfeat, vocab = wt_t.shape
    n = ids.shape[0]
    ncb = n // _CHT
    nk = (nfeat + _NW - 1) // _NW

    mesh = plsc.VectorSubcoreMesh(core_axis_name="c", subcore_axis_name="s")

    @functools.partial(
        pl.kernel,
        mesh=mesh,
        compiler_params=pltpu.CompilerParams(needs_layout_passes=False),
        out_type=jax.ShapeDtypeStruct((nfeat, n), jnp.float32),
        scratch_types=[
            pltpu.VMEM((vocab,), jnp.float32),
            pltpu.VMEM((_CHT,), jnp.int32),
            pltpu.VMEM((_CHT,), jnp.float32),
        ],
    )
    def k(wt_hbm, idx_hbm, out_hbm, row_v, ids_v, out_v):
        wid = lax.axis_index("s") * 2 + lax.axis_index("c")
        for kf in range(nk):
            f = wid + _NW * kf

            @pl.when(f < nfeat)
            def _():
                pltpu.sync_copy(wt_hbm.at[f], row_v)
                for cb in range(ncb):
                    pltpu.sync_copy(idx_hbm.at[pl.ds(cb * _CHT, _CHT)], ids_v)

                    @plsc.parallel_loop(0, _CHT, step=16, unroll=8)
                    def _gather(i):
                        idx = ids_v[pl.ds(i, 16)]
                        out_v[pl.ds(i, 16)] = plsc.load_gather(row_v, [idx])

                    pltpu.sync_copy(out_v,
                                    out_hbm.at[f, pl.ds(cb * _CHT, _CHT)])

    return k(wt_t, ids)


def _ln(x, w, b, dim):
    u = jnp.sum(x, axis=-1, keepdims=True) * (1.0 / dim)
    s = jnp.sum(x * x, axis=-1, keepdims=True) * (1.0 / dim) - u * u
    return w * ((x - u) * lax.rsqrt(s + _EPS)) + b


def _tc_body(wet_ref, vf_ref, tt_ref, wfcW_ref, vidW_ref, tok_ref, pe_ref,
             wcs, wc, wl2w, wl2b,
             vcs, vc, vl2w, vl2b,
             flw, flb, out_ref):
    xt = wet_ref[...]
    wvec = xt.shape[0]
    inv = 1.0 / wvec
    u_r = jnp.sum(xt, axis=0, keepdims=True) * inv
    s_r = jnp.sum(xt * xt, axis=0, keepdims=True) * inv - u_r * u_r
    u = u_r.reshape(-1, 1)
    s = s_r.reshape(-1, 1)
    mm = lax.dot_general(xt.astype(jnp.bfloat16), wfcW_ref[...],
                         (((0,), (0,)), ((), ())),
                         preferred_element_type=jnp.float32)
    hw = (mm - u * wcs[...]) * lax.rsqrt(s + _EPS) + wc[...]
    hw = _ln(jnp.maximum(hw, 0.0), wl2w[...], wl2b[...], 768.0)
    v = vf_ref[...]
    inv_v = 1.0 / v.shape[1]
    uv = jnp.sum(v, axis=-1, keepdims=True) * inv_v
    sv = jnp.sum(v * v, axis=-1, keepdims=True) * inv_v - uv * uv
    mv = jnp.dot(v.astype(jnp.bfloat16), vidW_ref[...],
                 preferred_element_type=jnp.float32)
    hv = (mv - uv * vcs[...]) * lax.rsqrt(sv + _EPS) + vc[...]
    hv = _ln(jnp.maximum(hv, 0.0), vl2w[...], vl2b[...], 768.0)
    ids = tt_ref[0]
    ids2 = ids.reshape(-1, 1)
    tt = jnp.where(ids2 == 0, tok_ref[0:1, :],
                   jnp.where(ids2 == 1, tok_ref[1:2, :], tok_ref[2:3, :]))
    emb = hw + hv + tt + pe_ref[...]
    out_ref[...] = _ln(emb, flw[...], flb[...], 768.0)


def _tc_fused(wet, vf, ttc, wfcW, vidW, tok_table, pe_t, vparams):
    wvec, n = wet.shape
    vid_feat = vf.shape[1]
    hid = vidW.shape[1]
    grid = n // _TB

    def tok_block(i):
        return (i, 0)

    def tok_cols(i):
        return (0, i)

    def tok_block3(i):
        return (i, 0, 0)

    def whole(i):
        return (0, 0)

    in_specs = [
        pl.BlockSpec((wvec, _TB), tok_cols),
        pl.BlockSpec((_TB, vid_feat), tok_block),
        pl.BlockSpec((1, 1, _TB), tok_block3),
        pl.BlockSpec(wfcW.shape, whole),
        pl.BlockSpec(vidW.shape, whole),
        pl.BlockSpec(tok_table.shape, whole),
        pl.BlockSpec(pe_t.shape, whole),
    ] + [pl.BlockSpec(p.shape, whole) for p in vparams]

    return pl.pallas_call(
        _tc_body,
        grid=(grid,),
        in_specs=in_specs,
        out_specs=pl.BlockSpec((_TB, hid), tok_block),
        out_shape=jax.ShapeDtypeStruct((n, hid), jnp.float32),
    )(wet, vf, ttc, wfcW, vidW, tok_table, pe_t, *vparams)


def kernel(input_ids, video_features, token_type_ids, word_table, tok_table,
           wfc_ln1_w, wfc_ln1_b, wfc_W, wfc_b, wfc_ln2_w, wfc_ln2_b,
           vid_ln1_w, vid_ln1_b, vid_W, vid_b, vid_ln2_w, vid_ln2_b,
           final_ln_w, final_ln_b, pe):
    b, l = input_ids.shape
    n = b * l
    hid = vid_W.shape[1]

    ids = input_ids.reshape(n).astype(jnp.int32)
    wet = _sc_gather_t(word_table.T, ids)

    vf = video_features.reshape(n, video_features.shape[2])
    ttc = token_type_ids.astype(jnp.int32).reshape(n // _TB, 1, _TB)
    pe_t = jnp.tile(pe[:l], (_TB // l, 1))

    row = lambda p: p.reshape(1, -1)
    wfcWs = wfc_ln1_w[:, None] * wfc_W
    w_colsum = row(jnp.sum(wfcWs, axis=0))
    w_const = row(wfc_ln1_b @ wfc_W + wfc_b)
    vidWs = vid_ln1_w[:, None] * vid_W
    v_colsum = row(jnp.sum(vidWs, axis=0))
    v_const = row(vid_ln1_b @ vid_W + vid_b)

    vparams = (w_colsum, w_const, row(wfc_ln2_w), row(wfc_ln2_b),
               v_colsum, v_const, row(vid_ln2_w), row(vid_ln2_b),
               row(final_ln_w), row(final_ln_b))

    out = _tc_fused(wet, vf, ttc,
                    wfcWs.astype(jnp.bfloat16), vidWs.astype(jnp.bfloat16),
                    tok_table, pe_t, vparams)
    return out.reshape(b, l, hid)


# speedup vs baseline: 3.3315x; 1.0246x over previous
import functools

import jax
import jax.numpy as jnp
from jax import lax
from jax.experimental import pallas as pl
from jax.experimental.pallas import tpu as pltpu
from jax.experimental.pallas import tpu_sc as plsc

_EPS = 1e-12
_TB = 256
_NW = 32
_CHT = 4096


def _sc_gather_t(wt_t, ids):
    nfeat, vocab = wt_t.shape
    n = ids.shape[0]
    ncb = n // _CHT
    nk = (nfeat + _NW - 1) // _NW

    mesh = plsc.VectorSubcoreMesh(core_axis_name="c", subcore_axis_name="s")

    @functools.partial(
        pl.kernel,
        mesh=mesh,
        compiler_params=pltpu.CompilerParams(needs_layout_passes=False),
        out_type=jax.ShapeDtypeStruct((nfeat, n), jnp.float32),
        scratch_types=[
            pltpu.VMEM((vocab,), jnp.float32),
            pltpu.VMEM((_CHT,), jnp.int32),
            pltpu.VMEM((_CHT,), jnp.int32),
            pltpu.VMEM((_CHT,), jnp.float32),
            pltpu.VMEM((_CHT,), jnp.float32),
            pltpu.SemaphoreType.DMA,
            pltpu.SemaphoreType.DMA,
            pltpu.SemaphoreType.DMA,
            pltpu.SemaphoreType.DMA,
        ],
    )
    def k(wt_hbm, idx_hbm, out_hbm, row_v, ids0, ids1, out0, out1,
          isem0, isem1, osem0, osem1):
        wid = lax.axis_index("s") * 2 + lax.axis_index("c")
        idbuf = (ids0, ids1)
        isem = (isem0, isem1)
        obuf = (out0, out1)
        osem = (osem0, osem1)
        for kf in range(nk):
            f = wid + _NW * kf

            @pl.when(f < nfeat)
            def _():
                pltpu.sync_copy(wt_hbm.at[f], row_v)
                pending = pltpu.async_copy(
                    idx_hbm.at[pl.ds(0, _CHT)], idbuf[0], isem[0])
                odrain = [None, None]
                for cb in range(ncb):
                    cur = pending
                    if cb + 1 < ncb:
                        pending = pltpu.async_copy(
                            idx_hbm.at[pl.ds((cb + 1) * _CHT, _CHT)],
                            idbuf[(cb + 1) % 2], isem[(cb + 1) % 2])
                    cur.wait()
                    ids_v = idbuf[cb % 2]
                    out_v = obuf[cb % 2]
                    if odrain[cb % 2] is not None:
                        odrain[cb % 2].wait()

                    @plsc.parallel_loop(0, _CHT, step=16, unroll=8)
                    def _gather(i):
                        idx = ids_v[pl.ds(i, 16)]
                        out_v[pl.ds(i, 16)] = plsc.load_gather(row_v, [idx])

                    odrain[cb % 2] = pltpu.async_copy(
                        out_v, out_hbm.at[f, pl.ds(cb * _CHT, _CHT)],
                        osem[cb % 2])
                for dr in odrain:
                    if dr is not None:
                        dr.wait()

    return k(wt_t, ids)


def _ln(x, w, b, dim):
    u = jnp.sum(x, axis=-1, keepdims=True) * (1.0 / dim)
    s = jnp.sum(x * x, axis=-1, keepdims=True) * (1.0 / dim) - u * u
    return w * ((x - u) * lax.rsqrt(s + _EPS)) + b


def _tc_video_body(vf_ref, tt_ref, vidW_ref, tok_ref, pe_ref,
                   vcs, vc, vl2w, vl2b, out_ref):
    v = vf_ref[...]
    inv_v = 1.0 / v.shape[1]
    uv = jnp.sum(v, axis=-1, keepdims=True) * inv_v
    sv = jnp.sum(v * v, axis=-1, keepdims=True) * inv_v - uv * uv
    mv = jnp.dot(v.astype(jnp.bfloat16), vidW_ref[...],
                 preferred_element_type=jnp.float32)
    hv = (mv - uv * vcs[...]) * lax.rsqrt(sv + _EPS) + vc[...]
    hv = _ln(jnp.maximum(hv, 0.0), vl2w[...], vl2b[...], 768.0)
    ids = tt_ref[0]
    ids2 = ids.reshape(-1, 1)
    tt = jnp.where(ids2 == 0, tok_ref[0:1, :],
                   jnp.where(ids2 == 1, tok_ref[1:2, :], tok_ref[2:3, :]))
    out_ref[...] = (hv + tt + pe_ref[...]).astype(jnp.bfloat16)


def _tc_word_body(wet_ref, part_ref, wfcW_ref,
                  wcs, wc, wl2w, wl2b, flw, flb, out_ref):
    xt = wet_ref[...]
    inv = 1.0 / xt.shape[0]
    u_r = jnp.sum(xt, axis=0, keepdims=True) * inv
    s_r = jnp.sum(xt * xt, axis=0, keepdims=True) * inv - u_r * u_r
    u = u_r.reshape(-1, 1)
    s = s_r.reshape(-1, 1)
    mm = lax.dot_general(xt.astype(jnp.bfloat16), wfcW_ref[...],
                         (((0,), (0,)), ((), ())),
                         preferred_element_type=jnp.float32)
    hw = (mm - u * wcs[...]) * lax.rsqrt(s + _EPS) + wc[...]
    hw = _ln(jnp.maximum(hw, 0.0), wl2w[...], wl2b[...], 768.0)
    emb = hw + part_ref[...].astype(jnp.float32)
    out_ref[...] = _ln(emb, flw[...], flb[...], 768.0)


def _whole(i):
    return (0, 0)


def _tok_block(i):
    return (i, 0)


def _tc_video(vf, ttc, vidW, tok_table, pe_t, params):
    n, vid_feat = vf.shape
    hid = vidW.shape[1]
    in_specs = [
        pl.BlockSpec((_TB, vid_feat), _tok_block),
        pl.BlockSpec((1, 1, _TB), lambda i: (i, 0, 0)),
        pl.BlockSpec(vidW.shape, _whole),
        pl.BlockSpec(tok_table.shape, _whole),
        pl.BlockSpec(pe_t.shape, _whole),
    ] + [pl.BlockSpec(p.shape, _whole) for p in params]
    return pl.pallas_call(
        _tc_video_body,
        grid=(n // _TB,),
        in_specs=in_specs,
        out_specs=pl.BlockSpec((_TB, hid), _tok_block),
        out_shape=jax.ShapeDtypeStruct((n, hid), jnp.bfloat16),
    )(vf, ttc, vidW, tok_table, pe_t, *params)


def _tc_word(wet, partial, wfcW, params):
    wvec, n = wet.shape
    hid = wfcW.shape[1]
    in_specs = [
        pl.BlockSpec((wvec, _TB), lambda i: (0, i)),
        pl.BlockSpec((_TB, hid), _tok_block),
        pl.BlockSpec(wfcW.shape, _whole),
    ] + [pl.BlockSpec(p.shape, _whole) for p in params]
    return pl.pallas_call(
        _tc_word_body,
        grid=(n // _TB,),
        in_specs=in_specs,
        out_specs=pl.BlockSpec((_TB, hid), _tok_block),
        out_shape=jax.ShapeDtypeStruct((n, hid), jnp.float32),
    )(wet, partial, wfcW, *params)


def kernel(input_ids, video_features, token_type_ids, word_table, tok_table,
           wfc_ln1_w, wfc_ln1_b, wfc_W, wfc_b, wfc_ln2_w, wfc_ln2_b,
           vid_ln1_w, vid_ln1_b, vid_W, vid_b, vid_ln2_w, vid_ln2_b,
           final_ln_w, final_ln_b, pe):
    b, l = input_ids.shape
    n = b * l
    hid = vid_W.shape[1]

    ids = input_ids.reshape(n).astype(jnp.int32)
    wet = _sc_gather_t(word_table.T, ids)

    vf = video_features.reshape(n, video_features.shape[2])
    ttc = token_type_ids.astype(jnp.int32).reshape(n // _TB, 1, _TB)
    pe_t = jnp.tile(pe[:l], (_TB // l, 1))

    row = lambda p: p.reshape(1, -1)
    wfcWs = wfc_ln1_w[:, None] * wfc_W
    w_colsum = row(jnp.sum(wfcWs, axis=0))
    w_const = row(wfc_ln1_b @ wfc_W + wfc_b)
    vidWs = vid_ln1_w[:, None] * vid_W
    v_colsum = row(jnp.sum(vidWs, axis=0))
    v_const = row(vid_ln1_b @ vid_W + vid_b)

    partial = _tc_video(vf, ttc, vidWs.astype(jnp.bfloat16), tok_table, pe_t,
                        (v_colsum, v_const, row(vid_ln2_w), row(vid_ln2_b)))
    out = _tc_word(wet, partial, wfcWs.astype(jnp.bfloat16),
                   (w_colsum, w_const, row(wfc_ln2_w), row(wfc_ln2_b),
                    row(final_ln_w), row(final_ln_b)))
    return out.reshape(b, l, hid)


# speedup vs baseline: 4.1855x; 1.2564x over previous
import functools

import jax
import jax.numpy as jnp
from jax import lax
from jax.experimental import pallas as pl
from jax.experimental.pallas import tpu as pltpu
from jax.experimental.pallas import tpu_sc as plsc

_EPS = 1e-12
_TB = 512
_NW = 32
_CHT = 4096


def _sc_gather_t(wt_t, ids):
    nfeat, vocab = wt_t.shape
    n = ids.shape[0]
    ncb = n // _CHT
    nk = (nfeat + _NW - 1) // _NW

    mesh = plsc.VectorSubcoreMesh(core_axis_name="c", subcore_axis_name="s")

    @functools.partial(
        pl.kernel,
        mesh=mesh,
        compiler_params=pltpu.CompilerParams(needs_layout_passes=False),
        out_type=jax.ShapeDtypeStruct((nfeat, n), jnp.float32),
        scratch_types=[
            pltpu.VMEM((vocab,), jnp.float32),
            pltpu.VMEM((_CHT,), jnp.int32),
            pltpu.VMEM((_CHT,), jnp.int32),
            pltpu.VMEM((_CHT,), jnp.float32),
            pltpu.VMEM((_CHT,), jnp.float32),
            pltpu.SemaphoreType.DMA,
            pltpu.SemaphoreType.DMA,
            pltpu.SemaphoreType.DMA,
            pltpu.SemaphoreType.DMA,
        ],
    )
    def k(wt_hbm, idx_hbm, out_hbm, row_v, ids0, ids1, out0, out1,
          isem0, isem1, osem0, osem1):
        wid = lax.axis_index("s") * 2 + lax.axis_index("c")
        idbuf = (ids0, ids1)
        isem = (isem0, isem1)
        obuf = (out0, out1)
        osem = (osem0, osem1)
        for kf in range(nk):
            f = wid + _NW * kf

            @pl.when(f < nfeat)
            def _():
                pltpu.sync_copy(wt_hbm.at[f], row_v)
                pending = pltpu.async_copy(
                    idx_hbm.at[pl.ds(0, _CHT)], idbuf[0], isem[0])
                odrain = [None, None]
                for cb in range(ncb):
                    cur = pending
                    if cb + 1 < ncb:
                        pending = pltpu.async_copy(
                            idx_hbm.at[pl.ds((cb + 1) * _CHT, _CHT)],
                            idbuf[(cb + 1) % 2], isem[(cb + 1) % 2])
                    cur.wait()
                    ids_v = idbuf[cb % 2]
                    out_v = obuf[cb % 2]
                    if odrain[cb % 2] is not None:
                        odrain[cb % 2].wait()

                    @plsc.parallel_loop(0, _CHT, step=16, unroll=8)
                    def _gather(i):
                        idx = ids_v[pl.ds(i, 16)]
                        out_v[pl.ds(i, 16)] = plsc.load_gather(row_v, [idx])

                    odrain[cb % 2] = pltpu.async_copy(
                        out_v, out_hbm.at[f, pl.ds(cb * _CHT, _CHT)],
                        osem[cb % 2])
                for dr in odrain:
                    if dr is not None:
                        dr.wait()

    return k(wt_t, ids)


def _ln(x, w, b, dim):
    u = jnp.sum(x, axis=-1, keepdims=True) * (1.0 / dim)
    s = jnp.sum(x * x, axis=-1, keepdims=True) * (1.0 / dim) - u * u
    return w * ((x - u) * lax.rsqrt(s + _EPS)) + b


def _tc_video_body(vf_ref, tt_ref, vidW_ref, tok_ref, pe_ref,
                   vcs, vc, vl2w, vl2b, out_ref):
    v = vf_ref[...]
    inv_v = 1.0 / v.shape[1]
    uv = jnp.sum(v, axis=-1, keepdims=True) * inv_v
    sv = jnp.sum(v * v, axis=-1, keepdims=True) * inv_v - uv * uv
    mv = jnp.dot(v.astype(jnp.bfloat16), vidW_ref[...],
                 preferred_element_type=jnp.float32)
    hv = (mv - uv * vcs[...]) * lax.rsqrt(sv + _EPS) + vc[...]
    hv = _ln(jnp.maximum(hv, 0.0), vl2w[...], vl2b[...], 768.0)
    ids = tt_ref[0]
    ids2 = ids.reshape(-1, 1)
    tt = jnp.where(ids2 == 0, tok_ref[0:1, :],
                   jnp.where(ids2 == 1, tok_ref[1:2, :], tok_ref[2:3, :]))
    out_ref[...] = (hv + tt + pe_ref[...]).astype(jnp.bfloat16)


def _tc_word_body(wet_ref, part_ref, wfcW_ref,
                  wcs, wc, wl2w, wl2b, flw, flb, out_ref):
    xt = wet_ref[...]
    inv = 1.0 / xt.shape[0]
    u_r = jnp.sum(xt, axis=0, keepdims=True) * inv
    s_r = jnp.sum(xt * xt, axis=0, keepdims=True) * inv - u_r * u_r
    u = u_r.reshape(-1, 1)
    s = s_r.reshape(-1, 1)
    mm = lax.dot_general(xt.astype(jnp.bfloat16), wfcW_ref[...],
                         (((0,), (0,)), ((), ())),
                         preferred_element_type=jnp.float32)
    hw = (mm - u * wcs[...]) * lax.rsqrt(s + _EPS) + wc[...]
    hw = _ln(jnp.maximum(hw, 0.0), wl2w[...], wl2b[...], 768.0)
    emb = hw + part_ref[...].astype(jnp.float32)
    out_ref[...] = _ln(emb, flw[...], flb[...], 768.0)


def _whole(i):
    return (0, 0)


def _tok_block(i):
    return (i, 0)


def _tc_video(vf, ttc, vidW, tok_table, pe_t, params):
    n, vid_feat = vf.shape
    hid = vidW.shape[1]
    in_specs = [
        pl.BlockSpec((_TB, vid_feat), _tok_block),
        pl.BlockSpec((1, 1, _TB), lambda i: (i, 0, 0)),
        pl.BlockSpec(vidW.shape, _whole),
        pl.BlockSpec(tok_table.shape, _whole),
        pl.BlockSpec(pe_t.shape, _whole),
    ] + [pl.BlockSpec(p.shape, _whole) for p in params]
    return pl.pallas_call(
        _tc_video_body,
        grid=(n // _TB,),
        in_specs=in_specs,
        out_specs=pl.BlockSpec((_TB, hid), _tok_block),
        out_shape=jax.ShapeDtypeStruct((n, hid), jnp.bfloat16),
    )(vf, ttc, vidW, tok_table, pe_t, *params)


def _tc_word(wet, partial, wfcW, params):
    wvec, n = wet.shape
    hid = wfcW.shape[1]
    in_specs = [
        pl.BlockSpec((wvec, _TB), lambda i: (0, i)),
        pl.BlockSpec((_TB, hid), _tok_block),
        pl.BlockSpec(wfcW.shape, _whole),
    ] + [pl.BlockSpec(p.shape, _whole) for p in params]
    return pl.pallas_call(
        _tc_word_body,
        grid=(n // _TB,),
        in_specs=in_specs,
        out_specs=pl.BlockSpec((_TB, hid), _tok_block),
        out_shape=jax.ShapeDtypeStruct((n, hid), jnp.float32),
    )(wet, partial, wfcW, *params)


def kernel(input_ids, video_features, token_type_ids, word_table, tok_table,
           wfc_ln1_w, wfc_ln1_b, wfc_W, wfc_b, wfc_ln2_w, wfc_ln2_b,
           vid_ln1_w, vid_ln1_b, vid_W, vid_b, vid_ln2_w, vid_ln2_b,
           final_ln_w, final_ln_b, pe):
    b, l = input_ids.shape
    n = b * l
    hid = vid_W.shape[1]

    ids = input_ids.reshape(n).astype(jnp.int32)
    wet = _sc_gather_t(word_table.T, ids)

    vf = video_features.reshape(n, video_features.shape[2])
    ttc = token_type_ids.astype(jnp.int32).reshape(n // _TB, 1, _TB)
    pe_t = jnp.tile(pe[:l], (_TB // l, 1))

    row = lambda p: p.reshape(1, -1)
    wfcWs = wfc_ln1_w[:, None] * wfc_W
    w_colsum = row(jnp.sum(wfcWs, axis=0))
    w_const = row(wfc_ln1_b @ wfc_W + wfc_b)
    vidWs = vid_ln1_w[:, None] * vid_W
    v_colsum = row(jnp.sum(vidWs, axis=0))
    v_const = row(vid_ln1_b @ vid_W + vid_b)

    partial = _tc_video(vf, ttc, vidWs.astype(jnp.bfloat16), tok_table, pe_t,
                        (v_colsum, v_const, row(vid_ln2_w), row(vid_ln2_b)))
    out = _tc_word(wet, partial, wfcWs.astype(jnp.bfloat16),
                   (w_colsum, w_const, row(wfc_ln2_w), row(wfc_ln2_b),
                    row(final_ln_w), row(final_ln_b)))
    return out.reshape(b, l, hid)


# speedup vs baseline: 4.3963x; 1.0504x over previous
import functools

import jax
import jax.numpy as jnp
from jax import lax
from jax.experimental import pallas as pl
from jax.experimental.pallas import tpu as pltpu
from jax.experimental.pallas import tpu_sc as plsc

_EPS = 1e-12
_TB = 1024
_NW = 32
_CHT = 4096


def _sc_gather_t(wt_t, ids):
    nfeat, vocab = wt_t.shape
    n = ids.shape[0]
    ncb = n // _CHT
    nk = (nfeat + _NW - 1) // _NW

    mesh = plsc.VectorSubcoreMesh(core_axis_name="c", subcore_axis_name="s")

    @functools.partial(
        pl.kernel,
        mesh=mesh,
        compiler_params=pltpu.CompilerParams(needs_layout_passes=False),
        out_type=jax.ShapeDtypeStruct((nfeat, n), jnp.float32),
        scratch_types=[
            pltpu.VMEM((vocab,), jnp.float32),
            pltpu.VMEM((_CHT,), jnp.int32),
            pltpu.VMEM((_CHT,), jnp.int32),
            pltpu.VMEM((_CHT,), jnp.float32),
            pltpu.VMEM((_CHT,), jnp.float32),
            pltpu.SemaphoreType.DMA,
            pltpu.SemaphoreType.DMA,
            pltpu.SemaphoreType.DMA,
            pltpu.SemaphoreType.DMA,
        ],
    )
    def k(wt_hbm, idx_hbm, out_hbm, row_v, ids0, ids1, out0, out1,
          isem0, isem1, osem0, osem1):
        wid = lax.axis_index("s") * 2 + lax.axis_index("c")
        idbuf = (ids0, ids1)
        isem = (isem0, isem1)
        obuf = (out0, out1)
        osem = (osem0, osem1)
        for kf in range(nk):
            f = wid + _NW * kf

            @pl.when(f < nfeat)
            def _():
                pltpu.sync_copy(wt_hbm.at[f], row_v)
                pending = pltpu.async_copy(
                    idx_hbm.at[pl.ds(0, _CHT)], idbuf[0], isem[0])
                odrain = [None, None]
                for cb in range(ncb):
                    cur = pending
                    if cb + 1 < ncb:
                        pending = pltpu.async_copy(
                            idx_hbm.at[pl.ds((cb + 1) * _CHT, _CHT)],
                            idbuf[(cb + 1) % 2], isem[(cb + 1) % 2])
                    cur.wait()
                    ids_v = idbuf[cb % 2]
                    out_v = obuf[cb % 2]
                    if odrain[cb % 2] is not None:
                        odrain[cb % 2].wait()

                    @plsc.parallel_loop(0, _CHT, step=16, unroll=8)
                    def _gather(i):
                        idx = ids_v[pl.ds(i, 16)]
                        out_v[pl.ds(i, 16)] = plsc.load_gather(row_v, [idx])

                    odrain[cb % 2] = pltpu.async_copy(
                        out_v, out_hbm.at[f, pl.ds(cb * _CHT, _CHT)],
                        osem[cb % 2])
                for dr in odrain:
                    if dr is not None:
                        dr.wait()

    return k(wt_t, ids)


def _ln(x, w, b, dim):
    u = jnp.sum(x, axis=-1, keepdims=True) * (1.0 / dim)
    s = jnp.sum(x * x, axis=-1, keepdims=True) * (1.0 / dim) - u * u
    return w * ((x - u) * lax.rsqrt(s + _EPS)) + b


def _tc_video_body(vf_ref, tt_ref, vidW_ref, tok_ref, pe_ref,
                   vcs, vc, vl2w, vl2b, out_ref):
    v = vf_ref[...]
    inv_v = 1.0 / v.shape[1]
    uv = jnp.sum(v, axis=-1, keepdims=True) * inv_v
    sv = jnp.sum(v * v, axis=-1, keepdims=True) * inv_v - uv * uv
    mv = jnp.dot(v.astype(jnp.bfloat16), vidW_ref[...],
                 preferred_element_type=jnp.float32)
    hv = (mv - uv * vcs[...]) * lax.rsqrt(sv + _EPS) + vc[...]
    hv = _ln(jnp.maximum(hv, 0.0), vl2w[...], vl2b[...], 768.0)
    ids = tt_ref[0]
    ids2 = ids.reshape(-1, 1)
    tt = jnp.where(ids2 == 0, tok_ref[0:1, :],
                   jnp.where(ids2 == 1, tok_ref[1:2, :], tok_ref[2:3, :]))
    out_ref[...] = (hv + tt + pe_ref[...]).astype(jnp.bfloat16)


def _tc_word_body(wet_ref, part_ref, wfcW_ref,
                  wcs, wc, wl2w, wl2b, flw, flb, out_ref):
    xt = wet_ref[...]
    inv = 1.0 / xt.shape[0]
    u_r = jnp.sum(xt, axis=0, keepdims=True) * inv
    s_r = jnp.sum(xt * xt, axis=0, keepdims=True) * inv - u_r * u_r
    u = u_r.reshape(-1, 1)
    s = s_r.reshape(-1, 1)
    mm = lax.dot_general(xt.astype(jnp.bfloat16), wfcW_ref[...],
                         (((0,), (0,)), ((), ())),
                         preferred_element_type=jnp.float32)
    hw = (mm - u * wcs[...]) * lax.rsqrt(s + _EPS) + wc[...]
    hw = _ln(jnp.maximum(hw, 0.0), wl2w[...], wl2b[...], 768.0)
    emb = hw + part_ref[...].astype(jnp.float32)
    out_ref[...] = _ln(emb, flw[...], flb[...], 768.0)


def _whole(i):
    return (0, 0)


def _tok_block(i):
    return (i, 0)


def _tc_video(vf, ttc, vidW, tok_table, pe_t, params):
    n, vid_feat = vf.shape
    hid = vidW.shape[1]
    in_specs = [
        pl.BlockSpec((_TB, vid_feat), _tok_block),
        pl.BlockSpec((1, 1, _TB), lambda i: (i, 0, 0)),
        pl.BlockSpec(vidW.shape, _whole),
        pl.BlockSpec(tok_table.shape, _whole),
        pl.BlockSpec(pe_t.shape, _whole),
    ] + [pl.BlockSpec(p.shape, _whole) for p in params]
    return pl.pallas_call(
        _tc_video_body,
        grid=(n // _TB,),
        in_specs=in_specs,
        out_specs=pl.BlockSpec((_TB, hid), _tok_block),
        out_shape=jax.ShapeDtypeStruct((n, hid), jnp.bfloat16),
    )(vf, ttc, vidW, tok_table, pe_t, *params)


def _tc_word(wet, partial, wfcW, params):
    wvec, n = wet.shape
    hid = wfcW.shape[1]
    in_specs = [
        pl.BlockSpec((wvec, _TB), lambda i: (0, i)),
        pl.BlockSpec((_TB, hid), _tok_block),
        pl.BlockSpec(wfcW.shape, _whole),
    ] + [pl.BlockSpec(p.shape, _whole) for p in params]
    return pl.pallas_call(
        _tc_word_body,
        grid=(n // _TB,),
        in_specs=in_specs,
        out_specs=pl.BlockSpec((_TB, hid), _tok_block),
        out_shape=jax.ShapeDtypeStruct((n, hid), jnp.float32),
    )(wet, partial, wfcW, *params)


def kernel(input_ids, video_features, token_type_ids, word_table, tok_table,
           wfc_ln1_w, wfc_ln1_b, wfc_W, wfc_b, wfc_ln2_w, wfc_ln2_b,
           vid_ln1_w, vid_ln1_b, vid_W, vid_b, vid_ln2_w, vid_ln2_b,
           final_ln_w, final_ln_b, pe):
    b, l = input_ids.shape
    n = b * l
    hid = vid_W.shape[1]

    ids = input_ids.reshape(n).astype(jnp.int32)
    wet = _sc_gather_t(word_table.T, ids)

    vf = video_features.reshape(n, video_features.shape[2])
    ttc = token_type_ids.astype(jnp.int32).reshape(n // _TB, 1, _TB)
    pe_t = jnp.tile(pe[:l], (_TB // l, 1))

    row = lambda p: p.reshape(1, -1)
    wfcWs = wfc_ln1_w[:, None] * wfc_W
    w_colsum = row(jnp.sum(wfcWs, axis=0))
    w_const = row(wfc_ln1_b @ wfc_W + wfc_b)
    vidWs = vid_ln1_w[:, None] * vid_W
    v_colsum = row(jnp.sum(vidWs, axis=0))
    v_const = row(vid_ln1_b @ vid_W + vid_b)

    partial = _tc_video(vf, ttc, vidWs.astype(jnp.bfloat16), tok_table, pe_t,
                        (v_colsum, v_const, row(vid_ln2_w), row(vid_ln2_b)))
    out = _tc_word(wet, partial, wfcWs.astype(jnp.bfloat16),
                   (w_colsum, w_const, row(wfc_ln2_w), row(wfc_ln2_b),
                    row(final_ln_w), row(final_ln_b)))
    return out.reshape(b, l, hid)
